# baseline (speedup 1.0000x reference)
import jax
import jax.numpy as jnp
from jax import lax
from jax.experimental import pallas as pl
from jax.experimental.pallas import tpu as pltpu
from jax.experimental.pallas import tpu_sc as plsc

N = 10000
E = 160000
ALPHA = 0.2

NC = 2
NS = 16
CHUNK = 128
E_PAD = 163840
NCH = E_PAD // CHUNK
ACC_ROWS = 10240
ROWS_PER_TILE = ACC_ROWS // NS
OUT_CHUNKS = ROWS_PER_TILE // CHUNK
RPT = NCH // (NC * NS)


def _make_seg_sum(edge_split):
  n_passes = 1 if edge_split else 2
  out_type = jax.ShapeDtypeStruct((NC * ACC_ROWS, 128), jnp.float32)

  mesh = plsc.VectorSubcoreMesh(
      core_axis_name="c", subcore_axis_name="s", num_cores=NC, num_subcores=NS)
  scratch = [
      pltpu.VMEM((RPT, CHUNK), jnp.int32),
      pltpu.VMEM((RPT, CHUNK), jnp.int32),
      pltpu.VMEM((CHUNK, 128), jnp.float32),
      pltpu.VMEM((CHUNK, 128), jnp.float32),
      pltpu.VMEM_SHARED((ACC_ROWS, 128), jnp.float32),
      pltpu.SemaphoreType.DMA,
      pltpu.SemaphoreType.DMA,
      pltpu.SemaphoreType.DMA,
      pltpu.SemaphoreType.DMA,
  ]

  def body(src_hbm, dst_hbm, table_hbm, z_hbm, out_hbm,
           sidx, didx, buf_a, buf_b, acc_s, gsem_a, gsem_b, ssem_a, ssem_b):
    cid = lax.axis_index("c")
    sid = lax.axis_index("s")
    row_base = sid * ROWS_PER_TILE

    pltpu.sync_copy(z_hbm, buf_a)
    for oc in range(OUT_CHUNKS):
      pltpu.sync_copy(buf_a, acc_s.at[pl.ds(row_base + oc * CHUNK, CHUNK)])
    plsc.subcore_barrier()

    bufs = (buf_a, buf_b)
    gsems = (gsem_a, gsem_b)
    ssems = (ssem_a, ssem_b)

    for p in range(n_passes):
      if edge_split:
        srow_base = (cid * NS + sid) * RPT
        drow_base = srow_base
      else:
        srow_base = cid * NCH + sid * (n_passes * RPT) + p * RPT
        drow_base = sid * (n_passes * RPT) + p * RPT
      pltpu.sync_copy(src_hbm.at[pl.ds(srow_base, RPT)], sidx)
      pltpu.sync_copy(dst_hbm.at[pl.ds(drow_base, RPT)], didx)

      def super_body(G, carry, p=p):
        for h in range(2):
          if p == 0:
            @pl.when(G > 0)
            def _(h=h):
              pltpu.make_async_copy(z_hbm, bufs[h], ssems[h]).wait()
          else:
            pltpu.make_async_copy(z_hbm, bufs[h], ssems[h]).wait()
          pltpu.async_copy(table_hbm.at[sidx.at[2 * G + h]], bufs[h],
                           gsems[h])
        for h in range(2):
          pltpu.make_async_copy(table_hbm.at[sidx.at[2 * G + h]], bufs[h],
                                gsems[h]).wait()
          pltpu.async_copy(bufs[h], acc_s.at[didx.at[2 * G + h]], ssems[h],
                           add=True)
        return carry

      lax.fori_loop(0, RPT // 2, super_body, 0)
    for h in range(2):
      pltpu.make_async_copy(z_hbm, bufs[h], ssems[h]).wait()
    plsc.subcore_barrier()

    out_base = cid * ACC_ROWS
    for oc in range(OUT_CHUNKS):
      r0 = row_base + oc * CHUNK
      pltpu.sync_copy(acc_s.at[pl.ds(r0, CHUNK)], bufs[oc % 2])
      pltpu.sync_copy(bufs[oc % 2], out_hbm.at[pl.ds(out_base + r0, CHUNK)])

  return pl.kernel(body, out_type=out_type, mesh=mesh, scratch_types=scratch)


def _make_deg_kernel():
  LAG = 8
  mesh = plsc.VectorSubcoreMesh(
      core_axis_name="c", subcore_axis_name="s", num_cores=NC, num_subcores=NS)
  out_type = jax.ShapeDtypeStruct((NC * ACC_ROWS, 128), jnp.float32)
  scratch = [
      pltpu.VMEM((RPT, CHUNK), jnp.int32),
      pltpu.VMEM((CHUNK, 128), jnp.float32),
      pltpu.VMEM((CHUNK, 128), jnp.float32),
      pltpu.VMEM_SHARED((ACC_ROWS, 128), jnp.float32),
      pltpu.SemaphoreType.DMA,
  ]

  def body(dst_hbm, ones_hbm, z2d_hbm, out_hbm, didx, ones_v, buf_v, acc_s,
           ssem):
    cid = lax.axis_index("c")
    sid = lax.axis_index("s")
    row_base = sid * ROWS_PER_TILE

    pltpu.sync_copy(dst_hbm.at[pl.ds((cid * NS + sid) * RPT, RPT)], didx)
    pltpu.sync_copy(ones_hbm, ones_v)
    pltpu.sync_copy(z2d_hbm, buf_v)
    for oc in range(OUT_CHUNKS):
      pltpu.sync_copy(buf_v, acc_s.at[pl.ds(row_base + oc * CHUNK, CHUNK)])
    plsc.subcore_barrier()

    def chunk_body(k, carry):
      pltpu.async_copy(ones_v, acc_s.at[didx.at[k]], ssem, add=True)

      @pl.when(k >= LAG)
      def _():
        pltpu.make_async_copy(z2d_hbm, buf_v, ssem).wait()
      return carry

    lax.fori_loop(0, RPT, chunk_body, 0)
    for _ in range(LAG):
      pltpu.make_async_copy(z2d_hbm, buf_v, ssem).wait()
    plsc.subcore_barrier()

    out_base = cid * ACC_ROWS
    for oc in range(OUT_CHUNKS):
      r0 = row_base + oc * CHUNK
      pltpu.sync_copy(acc_s.at[pl.ds(r0, CHUNK)], buf_v)
      pltpu.sync_copy(buf_v, out_hbm.at[pl.ds(out_base + r0, CHUNK)])

  return pl.kernel(body, out_type=out_type, mesh=mesh, scratch_types=scratch)


_sc_cache = {}


def _deg_kernel():
  if "deg" not in _sc_cache:
    _sc_cache["deg"] = _make_deg_kernel()
  return _sc_cache["deg"]


def _seg_sum(edge_split):
  key = ("seg", edge_split)
  if key not in _sc_cache:
    _sc_cache[key] = _make_seg_sum(edge_split)
  return _sc_cache[key]

_BM = 1000


def _tc1_body(x_ref, wl_ref, bl_ref, ws_ref, wn_ref, q1_ref, p1_ref):
  h = jnp.dot(x_ref[...], wl_ref[...], preferred_element_type=jnp.float32)
  h = h + bl_ref[...]
  h = jnp.where(h > 0, h, ALPHA * h)
  q1_ref[...] = jnp.dot(h, ws_ref[...], preferred_element_type=jnp.float32)
  p1_ref[...] = jnp.dot(h, wn_ref[...], preferred_element_type=jnp.float32)


def _tc1(x, W_lin, b_lin, Ws1, Wn1):
  grid = (N // _BM,)
  return pl.pallas_call(
      _tc1_body,
      grid=grid,
      in_specs=[
          pl.BlockSpec((_BM, 256), lambda i: (i, 0)),
          pl.BlockSpec((256, 256), lambda i: (0, 0)),
          pl.BlockSpec((1, 256), lambda i: (0, 0)),
          pl.BlockSpec((256, 256), lambda i: (0, 0)),
          pl.BlockSpec((256, 256), lambda i: (0, 0)),
      ],
      out_specs=[
          pl.BlockSpec((_BM, 256), lambda i: (i, 0)),
          pl.BlockSpec((_BM, 256), lambda i: (i, 0)),
      ],
      out_shape=[
          jax.ShapeDtypeStruct((N, 256), jnp.float32),
          jax.ShapeDtypeStruct((N, 256), jnp.float32),
      ],
  )(x, W_lin, b_lin.reshape(1, 256), Ws1, Wn1)


def _tc2_body(q1_ref, a1a_ref, a1b_ref, deg_ref, bc1_ref, ws2_ref, wn2_ref,
              q2_ref, p2_ref):
  inv = 1.0 / jnp.maximum(deg_ref[...], 1.0)
  agg = jnp.concatenate([a1a_ref[...], a1b_ref[...]], axis=1) * inv
  h = q1_ref[...] + agg + bc1_ref[...]
  h = jnp.maximum(h, 0.0)
  q2_ref[...] = jnp.dot(h, ws2_ref[...], preferred_element_type=jnp.float32)
  p2_ref[...] = jnp.dot(h, wn2_ref[...], preferred_element_type=jnp.float32)


def _tc2(q1, a1a, a1b, deg2d, bc1, Ws2, Wn2):
  grid = (N // _BM,)
  return pl.pallas_call(
      _tc2_body,
      grid=grid,
      in_specs=[
          pl.BlockSpec((_BM, 256), lambda i: (i, 0)),
          pl.BlockSpec((_BM, 128), lambda i: (i, 0)),
          pl.BlockSpec((_BM, 128), lambda i: (i, 0)),
          pl.BlockSpec((_BM, 1), lambda i: (i, 0)),
          pl.BlockSpec((1, 256), lambda i: (0, 0)),
          pl.BlockSpec((256, 128), lambda i: (0, 0)),
          pl.BlockSpec((256, 128), lambda i: (0, 0)),
      ],
      out_specs=[
          pl.BlockSpec((_BM, 128), lambda i: (i, 0)),
          pl.BlockSpec((_BM, 128), lambda i: (i, 0)),
      ],
      out_shape=[
          jax.ShapeDtypeStruct((N, 128), jnp.float32),
          jax.ShapeDtypeStruct((N, 128), jnp.float32),
      ],
  )(q1, a1a, a1b, deg2d, bc1.reshape(1, 256), Ws2, Wn2)


def _tc3_body(q2_ref, a2a_ref, a2b_ref, deg_ref, bc2_ref, wo_ref, bo_ref,
              out_ref):
  inv = 1.0 / jnp.maximum(deg_ref[...], 1.0)
  agg = (a2a_ref[...] + a2b_ref[...]) * inv
  h = q2_ref[...] + agg + bc2_ref[...]
  h = jnp.maximum(h, 0.0)
  out_ref[...] = jnp.dot(h, wo_ref[...], preferred_element_type=jnp.float32) + bo_ref[...]


def _tc3(q2, a2a, a2b, deg2d, bc2, W_out, b_out):
  grid = (N // _BM,)
  return pl.pallas_call(
      _tc3_body,
      grid=grid,
      in_specs=[
          pl.BlockSpec((_BM, 128), lambda i: (i, 0)),
          pl.BlockSpec((_BM, 128), lambda i: (i, 0)),
          pl.BlockSpec((_BM, 128), lambda i: (i, 0)),
          pl.BlockSpec((_BM, 1), lambda i: (i, 0)),
          pl.BlockSpec((1, 128), lambda i: (0, 0)),
          pl.BlockSpec((128, 1), lambda i: (0, 0)),
          pl.BlockSpec((1, 1), lambda i: (0, 0)),
      ],
      out_specs=pl.BlockSpec((_BM, 1), lambda i: (i, 0)),
      out_shape=jax.ShapeDtypeStruct((N, 1), jnp.float32),
  )(q2, a2a, a2b, deg2d, bc2.reshape(1, 128), W_out, b_out.reshape(1, 1))


def kernel(x, adj, edge_index, W_lin, b_lin, Ws1, Wn1, bc1, Ws2, Wn2, bc2,
           W_out, b_out):
  src = edge_index[0]
  dst = edge_index[1]
  pad = E_PAD - E
  src2d = jnp.concatenate([src, jnp.zeros((pad,), jnp.int32)]).reshape(
      NCH, CHUNK)
  dst2d = jnp.concatenate([dst, jnp.full((pad,), N, jnp.int32)]).reshape(
      NCH, CHUNK)
  src_cat = jnp.concatenate([src2d, src2d + N], axis=0)
  z2d_f32 = jnp.zeros((CHUNK, 128), jnp.float32)
  ones128 = jnp.ones((CHUNK, 128), jnp.float32)

  degf = _deg_kernel()(dst2d, ones128, z2d_f32)
  deg2d = degf[:N, 0:1] + degf[ACC_ROWS:ACC_ROWS + N, 0:1]

  q1, p1 = _tc1(x, W_lin, b_lin, Ws1, Wn1)
  table1 = jnp.concatenate([p1[:, :128], p1[:, 128:]], axis=0)
  a1f = _seg_sum(False)(src_cat, dst2d, table1, z2d_f32)
  a1a = a1f[:N]
  a1b = a1f[ACC_ROWS:ACC_ROWS + N]

  q2, p2 = _tc2(q1, a1a, a1b, deg2d, bc1, Ws2, Wn2)
  a2f = _seg_sum(True)(src2d, dst2d, p2, z2d_f32)
  a2a = a2f[:N]
  a2b = a2f[ACC_ROWS:ACC_ROWS + N]

  return _tc3(q2, a2a, a2b, deg2d, bc2, W_out, b_out)


# speedup vs baseline: 1.0057x; 1.0057x over previous
import jax
import jax.numpy as jnp
from jax import lax
from jax.experimental import pallas as pl
from jax.experimental.pallas import tpu as pltpu
from jax.experimental.pallas import tpu_sc as plsc

N = 10000
E = 160000
ALPHA = 0.2

NC = 2
NS = 16
CHUNK = 128
E_PAD = 163840
NCH = E_PAD // CHUNK
ACC_ROWS = 10240
ROWS_PER_TILE = ACC_ROWS // NS
OUT_CHUNKS = ROWS_PER_TILE // CHUNK
RPT = NCH // (NC * NS)


def _make_seg_sum(edge_split):
  n_passes = 1 if edge_split else 2
  out_type = jax.ShapeDtypeStruct((NC * ACC_ROWS, 128), jnp.float32)

  mesh = plsc.VectorSubcoreMesh(
      core_axis_name="c", subcore_axis_name="s", num_cores=NC, num_subcores=NS)
  scratch = [
      pltpu.VMEM((RPT, CHUNK), jnp.int32),
      pltpu.VMEM((RPT, CHUNK), jnp.int32),
      pltpu.VMEM((CHUNK, 128), jnp.float32),
      pltpu.VMEM((CHUNK, 128), jnp.float32),
      pltpu.VMEM_SHARED((ACC_ROWS, 128), jnp.float32),
      pltpu.SemaphoreType.DMA,
      pltpu.SemaphoreType.DMA,
      pltpu.SemaphoreType.DMA,
      pltpu.SemaphoreType.DMA,
  ]

  def body(src_hbm, dst_hbm, table_hbm, z_hbm, out_hbm,
           sidx, didx, buf_a, buf_b, acc_s, gsem_a, gsem_b, ssem_a, ssem_b):
    cid = lax.axis_index("c")
    sid = lax.axis_index("s")
    row_base = sid * ROWS_PER_TILE

    pltpu.sync_copy(z_hbm, buf_a)
    for oc in range(OUT_CHUNKS):
      pltpu.sync_copy(buf_a, acc_s.at[pl.ds(row_base + oc * CHUNK, CHUNK)])
    plsc.subcore_barrier()

    bufs = (buf_a, buf_b)
    gsems = (gsem_a, gsem_b)
    ssems = (ssem_a, ssem_b)

    for p in range(n_passes):
      if edge_split:
        srow_base = (cid * NS + sid) * RPT
        drow_base = srow_base
      else:
        srow_base = cid * NCH + sid * (n_passes * RPT) + p * RPT
        drow_base = sid * (n_passes * RPT) + p * RPT
      pltpu.sync_copy(src_hbm.at[pl.ds(srow_base, RPT)], sidx)
      pltpu.sync_copy(dst_hbm.at[pl.ds(drow_base, RPT)], didx)

      def super_body(G, carry, p=p):
        for h in range(2):
          g = 2 * G + h
          if p == 0:
            @pl.when(g > 1)
            def _(h=h):
              pltpu.make_async_copy(z_hbm, bufs[h], ssems[h]).wait()
          else:
            pltpu.make_async_copy(z_hbm, bufs[h], ssems[h]).wait()
          pltpu.async_copy(table_hbm.at[sidx.at[g]], bufs[h], gsems[h]).wait()
          pltpu.async_copy(bufs[h], acc_s.at[didx.at[g]], ssems[h], add=True)
        return carry

      lax.fori_loop(0, RPT // 2, super_body, 0)
    for h in range(2):
      pltpu.make_async_copy(z_hbm, bufs[h], ssems[h]).wait()
    plsc.subcore_barrier()

    out_base = cid * ACC_ROWS
    for oc in range(OUT_CHUNKS):
      r0 = row_base + oc * CHUNK
      pltpu.sync_copy(acc_s.at[pl.ds(r0, CHUNK)], bufs[oc % 2])
      pltpu.sync_copy(bufs[oc % 2], out_hbm.at[pl.ds(out_base + r0, CHUNK)])

  return pl.kernel(body, out_type=out_type, mesh=mesh, scratch_types=scratch)


def _make_deg_kernel():
  LAG = 8
  mesh = plsc.VectorSubcoreMesh(
      core_axis_name="c", subcore_axis_name="s", num_cores=NC, num_subcores=NS)
  out_type = jax.ShapeDtypeStruct((NC * ACC_ROWS, 128), jnp.float32)
  scratch = [
      pltpu.VMEM((RPT, CHUNK), jnp.int32),
      pltpu.VMEM((CHUNK, 128), jnp.float32),
      pltpu.VMEM((CHUNK, 128), jnp.float32),
      pltpu.VMEM_SHARED((ACC_ROWS, 128), jnp.float32),
      pltpu.SemaphoreType.DMA,
  ]

  def body(dst_hbm, ones_hbm, z2d_hbm, out_hbm, didx, ones_v, buf_v, acc_s,
           ssem):
    cid = lax.axis_index("c")
    sid = lax.axis_index("s")
    row_base = sid * ROWS_PER_TILE

    pltpu.sync_copy(dst_hbm.at[pl.ds((cid * NS + sid) * RPT, RPT)], didx)
    pltpu.sync_copy(ones_hbm, ones_v)
    pltpu.sync_copy(z2d_hbm, buf_v)
    for oc in range(OUT_CHUNKS):
      pltpu.sync_copy(buf_v, acc_s.at[pl.ds(row_base + oc * CHUNK, CHUNK)])
    plsc.subcore_barrier()

    def chunk_body(k, carry):
      pltpu.async_copy(ones_v, acc_s.at[didx.at[k]], ssem, add=True)

      @pl.when(k >= LAG)
      def _():
        pltpu.make_async_copy(z2d_hbm, buf_v, ssem).wait()
      return carry

    lax.fori_loop(0, RPT, chunk_body, 0)
    for _ in range(LAG):
      pltpu.make_async_copy(z2d_hbm, buf_v, ssem).wait()
    plsc.subcore_barrier()

    out_base = cid * ACC_ROWS
    for oc in range(OUT_CHUNKS):
      r0 = row_base + oc * CHUNK
      pltpu.sync_copy(acc_s.at[pl.ds(r0, CHUNK)], buf_v)
      pltpu.sync_copy(buf_v, out_hbm.at[pl.ds(out_base + r0, CHUNK)])

  return pl.kernel(body, out_type=out_type, mesh=mesh, scratch_types=scratch)


_sc_cache = {}


def _deg_kernel():
  if "deg" not in _sc_cache:
    _sc_cache["deg"] = _make_deg_kernel()
  return _sc_cache["deg"]


def _seg_sum(edge_split):
  key = ("seg", edge_split)
  if key not in _sc_cache:
    _sc_cache[key] = _make_seg_sum(edge_split)
  return _sc_cache[key]

_BM = 1000


def _tc1_body(x_ref, wl_ref, bl_ref, ws_ref, wn_ref, q1_ref, p1_ref):
  h = jnp.dot(x_ref[...], wl_ref[...], preferred_element_type=jnp.float32)
  h = h + bl_ref[...]
  h = jnp.where(h > 0, h, ALPHA * h)
  q1_ref[...] = jnp.dot(h, ws_ref[...], preferred_element_type=jnp.float32)
  p1_ref[...] = jnp.dot(h, wn_ref[...], preferred_element_type=jnp.float32)


def _tc1(x, W_lin, b_lin, Ws1, Wn1):
  grid = (N // _BM,)
  return pl.pallas_call(
      _tc1_body,
      grid=grid,
      in_specs=[
          pl.BlockSpec((_BM, 256), lambda i: (i, 0)),
          pl.BlockSpec((256, 256), lambda i: (0, 0)),
          pl.BlockSpec((1, 256), lambda i: (0, 0)),
          pl.BlockSpec((256, 256), lambda i: (0, 0)),
          pl.BlockSpec((256, 256), lambda i: (0, 0)),
      ],
      out_specs=[
          pl.BlockSpec((_BM, 256), lambda i: (i, 0)),
          pl.BlockSpec((_BM, 256), lambda i: (i, 0)),
      ],
      out_shape=[
          jax.ShapeDtypeStruct((N, 256), jnp.float32),
          jax.ShapeDtypeStruct((N, 256), jnp.float32),
      ],
  )(x, W_lin, b_lin.reshape(1, 256), Ws1, Wn1)


def _tc2_body(q1_ref, a1a_ref, a1b_ref, deg_ref, bc1_ref, ws2_ref, wn2_ref,
              q2_ref, p2_ref):
  inv = 1.0 / jnp.maximum(deg_ref[...], 1.0)
  agg = jnp.concatenate([a1a_ref[...], a1b_ref[...]], axis=1) * inv
  h = q1_ref[...] + agg + bc1_ref[...]
  h = jnp.maximum(h, 0.0)
  q2_ref[...] = jnp.dot(h, ws2_ref[...], preferred_element_type=jnp.float32)
  p2_ref[...] = jnp.dot(h, wn2_ref[...], preferred_element_type=jnp.float32)


def _tc2(q1, a1a, a1b, deg2d, bc1, Ws2, Wn2):
  grid = (N // _BM,)
  return pl.pallas_call(
      _tc2_body,
      grid=grid,
      in_specs=[
          pl.BlockSpec((_BM, 256), lambda i: (i, 0)),
          pl.BlockSpec((_BM, 128), lambda i: (i, 0)),
          pl.BlockSpec((_BM, 128), lambda i: (i, 0)),
          pl.BlockSpec((_BM, 1), lambda i: (i, 0)),
          pl.BlockSpec((1, 256), lambda i: (0, 0)),
          pl.BlockSpec((256, 128), lambda i: (0, 0)),
          pl.BlockSpec((256, 128), lambda i: (0, 0)),
      ],
      out_specs=[
          pl.BlockSpec((_BM, 128), lambda i: (i, 0)),
          pl.BlockSpec((_BM, 128), lambda i: (i, 0)),
      ],
      out_shape=[
          jax.ShapeDtypeStruct((N, 128), jnp.float32),
          jax.ShapeDtypeStruct((N, 128), jnp.float32),
      ],
  )(q1, a1a, a1b, deg2d, bc1.reshape(1, 256), Ws2, Wn2)


def _tc3_body(q2_ref, a2a_ref, a2b_ref, deg_ref, bc2_ref, wo_ref, bo_ref,
              out_ref):
  inv = 1.0 / jnp.maximum(deg_ref[...], 1.0)
  agg = (a2a_ref[...] + a2b_ref[...]) * inv
  h = q2_ref[...] + agg + bc2_ref[...]
  h = jnp.maximum(h, 0.0)
  out_ref[...] = jnp.dot(h, wo_ref[...], preferred_element_type=jnp.float32) + bo_ref[...]


def _tc3(q2, a2a, a2b, deg2d, bc2, W_out, b_out):
  grid = (N // _BM,)
  return pl.pallas_call(
      _tc3_body,
      grid=grid,
      in_specs=[
          pl.BlockSpec((_BM, 128), lambda i: (i, 0)),
          pl.BlockSpec((_BM, 128), lambda i: (i, 0)),
          pl.BlockSpec((_BM, 128), lambda i: (i, 0)),
          pl.BlockSpec((_BM, 1), lambda i: (i, 0)),
          pl.BlockSpec((1, 128), lambda i: (0, 0)),
          pl.BlockSpec((128, 1), lambda i: (0, 0)),
          pl.BlockSpec((1, 1), lambda i: (0, 0)),
      ],
      out_specs=pl.BlockSpec((_BM, 1), lambda i: (i, 0)),
      out_shape=jax.ShapeDtypeStruct((N, 1), jnp.float32),
  )(q2, a2a, a2b, deg2d, bc2.reshape(1, 128), W_out, b_out.reshape(1, 1))


def kernel(x, adj, edge_index, W_lin, b_lin, Ws1, Wn1, bc1, Ws2, Wn2, bc2,
           W_out, b_out):
  src = edge_index[0]
  dst = edge_index[1]
  pad = E_PAD - E
  src2d = jnp.concatenate([src, jnp.zeros((pad,), jnp.int32)]).reshape(
      NCH, CHUNK)
  dst2d = jnp.concatenate([dst, jnp.full((pad,), N, jnp.int32)]).reshape(
      NCH, CHUNK)
  src_cat = jnp.concatenate([src2d, src2d + N], axis=0)
  z2d_f32 = jnp.zeros((CHUNK, 128), jnp.float32)
  ones128 = jnp.ones((CHUNK, 128), jnp.float32)

  degf = _deg_kernel()(dst2d, ones128, z2d_f32)
  deg2d = degf[:N, 0:1] + degf[ACC_ROWS:ACC_ROWS + N, 0:1]

  q1, p1 = _tc1(x, W_lin, b_lin, Ws1, Wn1)
  table1 = jnp.concatenate([p1[:, :128], p1[:, 128:]], axis=0)
  a1f = _seg_sum(False)(src_cat, dst2d, table1, z2d_f32)
  a1a = a1f[:N]
  a1b = a1f[ACC_ROWS:ACC_ROWS + N]

  q2, p2 = _tc2(q1, a1a, a1b, deg2d, bc1, Ws2, Wn2)
  a2f = _seg_sum(True)(src2d, dst2d, p2, z2d_f32)
  a2a = a2f[:N]
  a2b = a2f[ACC_ROWS:ACC_ROWS + N]

  return _tc3(q2, a2a, a2b, deg2d, bc2, W_out, b_out)


# speedup vs baseline: 1.0166x; 1.0108x over previous
import jax
import jax.numpy as jnp
from jax import lax
from jax.experimental import pallas as pl
from jax.experimental.pallas import tpu as pltpu
from jax.experimental.pallas import tpu_sc as plsc

N = 10000
E = 160000
ALPHA = 0.2

NC = 2
NS = 16
CHUNK = 128
E_PAD = 163840
NCH = E_PAD // CHUNK
ACC_ROWS = 10240
ROWS_PER_TILE = ACC_ROWS // NS
OUT_CHUNKS = ROWS_PER_TILE // CHUNK
RPT = NCH // (NC * NS)


def _make_seg_sum(edge_split):
  n_passes = 1 if edge_split else 2
  out_type = jax.ShapeDtypeStruct((NC * ACC_ROWS, 128), jnp.float32)

  mesh = plsc.VectorSubcoreMesh(
      core_axis_name="c", subcore_axis_name="s", num_cores=NC, num_subcores=NS)
  scratch = [
      pltpu.VMEM((RPT, CHUNK), jnp.int32),
      pltpu.VMEM((RPT, CHUNK), jnp.int32),
      pltpu.VMEM((CHUNK, 128), jnp.float32),
      pltpu.VMEM((CHUNK, 128), jnp.float32),
      pltpu.VMEM_SHARED((ACC_ROWS, 128), jnp.float32),
      pltpu.SemaphoreType.DMA,
      pltpu.SemaphoreType.DMA,
      pltpu.SemaphoreType.DMA,
      pltpu.SemaphoreType.DMA,
  ]

  def body(src_hbm, dst_hbm, table_hbm, z_hbm, out_hbm,
           sidx, didx, buf_a, buf_b, acc_s, gsem_a, gsem_b, ssem_a, ssem_b):
    cid = lax.axis_index("c")
    sid = lax.axis_index("s")
    row_base = sid * ROWS_PER_TILE

    pltpu.sync_copy(z_hbm, buf_a)
    for oc in range(OUT_CHUNKS):
      pltpu.sync_copy(buf_a, acc_s.at[pl.ds(row_base + oc * CHUNK, CHUNK)])
    plsc.subcore_barrier()

    bufs = (buf_a, buf_b)
    gsems = (gsem_a, gsem_b)
    ssems = (ssem_a, ssem_b)

    for p in range(n_passes):
      if edge_split:
        srow_base = (cid * NS + sid) * RPT
        drow_base = srow_base
      else:
        srow_base = cid * NCH + sid * (n_passes * RPT) + p * RPT
        drow_base = sid * (n_passes * RPT) + p * RPT
      pltpu.sync_copy(src_hbm.at[pl.ds(srow_base, RPT)], sidx)
      pltpu.sync_copy(dst_hbm.at[pl.ds(drow_base, RPT)], didx)

      def super_body(G, carry, p=p):
        for h in range(2):
          g = 2 * G + h
          if p == 0:
            @pl.when(g > 1)
            def _(h=h):
              pltpu.make_async_copy(z_hbm, bufs[h], ssems[h]).wait()
          else:
            pltpu.make_async_copy(z_hbm, bufs[h], ssems[h]).wait()
          pltpu.async_copy(table_hbm.at[sidx.at[g]], bufs[h], gsems[h]).wait()
          pltpu.async_copy(bufs[h], acc_s.at[didx.at[g]], ssems[h], add=True)
        return carry

      lax.fori_loop(0, RPT // 2, super_body, 0)
    for h in range(2):
      pltpu.make_async_copy(z_hbm, bufs[h], ssems[h]).wait()
    plsc.subcore_barrier()

    out_base = cid * ACC_ROWS
    for oc in range(OUT_CHUNKS):
      r0 = row_base + oc * CHUNK
      pltpu.sync_copy(acc_s.at[pl.ds(r0, CHUNK)], bufs[oc % 2])
      pltpu.sync_copy(bufs[oc % 2], out_hbm.at[pl.ds(out_base + r0, CHUNK)])

  return pl.kernel(body, out_type=out_type, mesh=mesh, scratch_types=scratch)


def _make_deg_kernel():
  LAG = 8
  mesh = plsc.VectorSubcoreMesh(
      core_axis_name="c", subcore_axis_name="s", num_cores=NC, num_subcores=NS)
  out_type = jax.ShapeDtypeStruct((NC * ACC_ROWS, 128), jnp.float32)
  scratch = [
      pltpu.VMEM((RPT, CHUNK), jnp.int32),
      pltpu.VMEM((CHUNK, 128), jnp.float32),
      pltpu.VMEM((CHUNK, 128), jnp.float32),
      pltpu.VMEM_SHARED((ACC_ROWS, 128), jnp.float32),
      pltpu.SemaphoreType.DMA,
  ]

  def body(dst_hbm, ones_hbm, z2d_hbm, out_hbm, didx, ones_v, buf_v, acc_s,
           ssem):
    cid = lax.axis_index("c")
    sid = lax.axis_index("s")
    row_base = sid * ROWS_PER_TILE

    pltpu.sync_copy(dst_hbm.at[pl.ds((cid * NS + sid) * RPT, RPT)], didx)
    pltpu.sync_copy(ones_hbm, ones_v)
    pltpu.sync_copy(z2d_hbm, buf_v)
    for oc in range(OUT_CHUNKS):
      pltpu.sync_copy(buf_v, acc_s.at[pl.ds(row_base + oc * CHUNK, CHUNK)])
    plsc.subcore_barrier()

    def chunk_body(k, carry):
      pltpu.async_copy(ones_v, acc_s.at[didx.at[k]], ssem, add=True)

      @pl.when(k >= LAG)
      def _():
        pltpu.make_async_copy(z2d_hbm, buf_v, ssem).wait()
      return carry

    lax.fori_loop(0, RPT, chunk_body, 0)
    for _ in range(LAG):
      pltpu.make_async_copy(z2d_hbm, buf_v, ssem).wait()
    plsc.subcore_barrier()

    out_base = cid * ACC_ROWS
    for oc in range(OUT_CHUNKS):
      r0 = row_base + oc * CHUNK
      pltpu.sync_copy(acc_s.at[pl.ds(r0, CHUNK)], buf_v)
      pltpu.sync_copy(buf_v, out_hbm.at[pl.ds(out_base + r0, CHUNK)])

  return pl.kernel(body, out_type=out_type, mesh=mesh, scratch_types=scratch)


_sc_cache = {}


def _deg_kernel():
  if "deg" not in _sc_cache:
    _sc_cache["deg"] = _make_deg_kernel()
  return _sc_cache["deg"]


def _seg_sum(edge_split):
  key = ("seg", edge_split)
  if key not in _sc_cache:
    _sc_cache[key] = _make_seg_sum(edge_split)
  return _sc_cache[key]

_BM = 2000


def _tc1_body(x_ref, wl_ref, bl_ref, ws_ref, wn_ref, q1_ref, p1_ref):
  h = jnp.dot(x_ref[...], wl_ref[...], preferred_element_type=jnp.float32)
  h = h + bl_ref[...]
  h = jnp.where(h > 0, h, ALPHA * h)
  q1_ref[...] = jnp.dot(h, ws_ref[...], preferred_element_type=jnp.float32)
  p1_ref[...] = jnp.dot(h, wn_ref[...], preferred_element_type=jnp.float32)


def _tc1(x, W_lin, b_lin, Ws1, Wn1):
  grid = (N // _BM,)
  return pl.pallas_call(
      _tc1_body,
      grid=grid,
      in_specs=[
          pl.BlockSpec((_BM, 256), lambda i: (i, 0)),
          pl.BlockSpec((256, 256), lambda i: (0, 0)),
          pl.BlockSpec((1, 256), lambda i: (0, 0)),
          pl.BlockSpec((256, 256), lambda i: (0, 0)),
          pl.BlockSpec((256, 256), lambda i: (0, 0)),
      ],
      out_specs=[
          pl.BlockSpec((_BM, 256), lambda i: (i, 0)),
          pl.BlockSpec((_BM, 256), lambda i: (i, 0)),
      ],
      out_shape=[
          jax.ShapeDtypeStruct((N, 256), jnp.float32),
          jax.ShapeDtypeStruct((N, 256), jnp.float32),
      ],
  )(x, W_lin, b_lin.reshape(1, 256), Ws1, Wn1)


def _tc2_body(q1_ref, a1a_ref, a1b_ref, deg_ref, bc1_ref, ws2_ref, wn2_ref,
              q2_ref, p2_ref):
  inv = 1.0 / jnp.maximum(deg_ref[...], 1.0)
  agg = jnp.concatenate([a1a_ref[...], a1b_ref[...]], axis=1) * inv
  h = q1_ref[...] + agg + bc1_ref[...]
  h = jnp.maximum(h, 0.0)
  q2_ref[...] = jnp.dot(h, ws2_ref[...], preferred_element_type=jnp.float32)
  p2_ref[...] = jnp.dot(h, wn2_ref[...], preferred_element_type=jnp.float32)


def _tc2(q1, a1a, a1b, deg2d, bc1, Ws2, Wn2):
  grid = (N // _BM,)
  return pl.pallas_call(
      _tc2_body,
      grid=grid,
      in_specs=[
          pl.BlockSpec((_BM, 256), lambda i: (i, 0)),
          pl.BlockSpec((_BM, 128), lambda i: (i, 0)),
          pl.BlockSpec((_BM, 128), lambda i: (i, 0)),
          pl.BlockSpec((_BM, 1), lambda i: (i, 0)),
          pl.BlockSpec((1, 256), lambda i: (0, 0)),
          pl.BlockSpec((256, 128), lambda i: (0, 0)),
          pl.BlockSpec((256, 128), lambda i: (0, 0)),
      ],
      out_specs=[
          pl.BlockSpec((_BM, 128), lambda i: (i, 0)),
          pl.BlockSpec((_BM, 128), lambda i: (i, 0)),
      ],
      out_shape=[
          jax.ShapeDtypeStruct((N, 128), jnp.float32),
          jax.ShapeDtypeStruct((N, 128), jnp.float32),
      ],
  )(q1, a1a, a1b, deg2d, bc1.reshape(1, 256), Ws2, Wn2)


def _tc3_body(q2_ref, a2a_ref, a2b_ref, deg_ref, bc2_ref, wo_ref, bo_ref,
              out_ref):
  inv = 1.0 / jnp.maximum(deg_ref[...], 1.0)
  agg = (a2a_ref[...] + a2b_ref[...]) * inv
  h = q2_ref[...] + agg + bc2_ref[...]
  h = jnp.maximum(h, 0.0)
  out_ref[...] = jnp.dot(h, wo_ref[...], preferred_element_type=jnp.float32) + bo_ref[...]


def _tc3(q2, a2a, a2b, deg2d, bc2, W_out, b_out):
  grid = (N // _BM,)
  return pl.pallas_call(
      _tc3_body,
      grid=grid,
      in_specs=[
          pl.BlockSpec((_BM, 128), lambda i: (i, 0)),
          pl.BlockSpec((_BM, 128), lambda i: (i, 0)),
          pl.BlockSpec((_BM, 128), lambda i: (i, 0)),
          pl.BlockSpec((_BM, 1), lambda i: (i, 0)),
          pl.BlockSpec((1, 128), lambda i: (0, 0)),
          pl.BlockSpec((128, 1), lambda i: (0, 0)),
          pl.BlockSpec((1, 1), lambda i: (0, 0)),
      ],
      out_specs=pl.BlockSpec((_BM, 1), lambda i: (i, 0)),
      out_shape=jax.ShapeDtypeStruct((N, 1), jnp.float32),
  )(q2, a2a, a2b, deg2d, bc2.reshape(1, 128), W_out, b_out.reshape(1, 1))


def kernel(x, adj, edge_index, W_lin, b_lin, Ws1, Wn1, bc1, Ws2, Wn2, bc2,
           W_out, b_out):
  src = edge_index[0]
  dst = edge_index[1]
  pad = E_PAD - E
  src2d = jnp.concatenate([src, jnp.zeros((pad,), jnp.int32)]).reshape(
      NCH, CHUNK)
  dst2d = jnp.concatenate([dst, jnp.full((pad,), N, jnp.int32)]).reshape(
      NCH, CHUNK)
  src_cat = jnp.concatenate([src2d, src2d + N], axis=0)
  z2d_f32 = jnp.zeros((CHUNK, 128), jnp.float32)
  ones128 = jnp.ones((CHUNK, 128), jnp.float32)

  degf = _deg_kernel()(dst2d, ones128, z2d_f32)
  deg2d = degf[:N, 0:1] + degf[ACC_ROWS:ACC_ROWS + N, 0:1]

  q1, p1 = _tc1(x, W_lin, b_lin, Ws1, Wn1)
  table1 = jnp.concatenate([p1[:, :128], p1[:, 128:]], axis=0)
  a1f = _seg_sum(False)(src_cat, dst2d, table1, z2d_f32)
  a1a = a1f[:N]
  a1b = a1f[ACC_ROWS:ACC_ROWS + N]

  q2, p2 = _tc2(q1, a1a, a1b, deg2d, bc1, Ws2, Wn2)
  a2f = _seg_sum(True)(src2d, dst2d, p2, z2d_f32)
  a2a = a2f[:N]
  a2b = a2f[ACC_ROWS:ACC_ROWS + N]

  return _tc3(q2, a2a, a2b, deg2d, bc2, W_out, b_out)
